# baseline (speedup 1.0000x reference)
import functools

import jax
import jax.numpy as jnp
from jax import lax
from jax.experimental import pallas as pl
from jax.experimental.pallas import tpu as pltpu
from jax.experimental.pallas import tpu_sc as plsc

_B = 4
_T = 2048
_C = 64
_NC = 2
_NS = 16
_NW = _NC * _NS
_ROWS = (_B * _T) // _NW
_CHUNK = 128
_NCHUNK = _ROWS // _CHUNK


def _emb_body(x_hbm, tok_hbm, pos_hbm, out_hbm, idx_v, rows_v, pos_v, sem):
    wid = lax.axis_index("s") * _NC + lax.axis_index("c")
    base = wid * _ROWS

    pltpu.sync_copy(x_hbm.at[pl.ds(wid * _NCHUNK, _NCHUNK)], idx_v)

    copies = []
    for j in range(_NCHUNK):
        copies.append(
            pltpu.async_copy(
                tok_hbm.at[idx_v.at[j]],
                rows_v.at[pl.ds(j * _CHUNK, _CHUNK)],
                sem,
            )
        )

    t0 = lax.rem(base, _T)
    pltpu.sync_copy(pos_hbm.at[pl.ds(t0, _ROWS)], pos_v)

    for c in copies:
        c.wait()

    def add_row(i, carry):
        for j in range(_C // 16):
            sl = pl.ds(j * 16, 16)
            rows_v[i, sl] = rows_v[i, sl] + pos_v[i, sl]
        return carry

    lax.fori_loop(0, _ROWS, add_row, 0, unroll=4)

    pltpu.sync_copy(rows_v, out_hbm.at[pl.ds(base, _ROWS)])


@jax.jit
def _emb(x2d, tok_table, pos_table):
    mesh = plsc.VectorSubcoreMesh(core_axis_name="c", subcore_axis_name="s")
    f = pl.kernel(
        _emb_body,
        out_type=jax.ShapeDtypeStruct((_B * _T, _C), jnp.float32),
        mesh=mesh,
        scratch_types=[
            pltpu.VMEM((_NCHUNK, _CHUNK), jnp.int32),
            pltpu.VMEM((_ROWS, _C), jnp.float32),
            pltpu.VMEM((_ROWS, _C), jnp.float32),
            pltpu.SemaphoreType.DMA,
        ],
        compiler_params=pltpu.CompilerParams(use_tc_tiling_on_sc=False),
    )
    return f(x2d, tok_table, pos_table)


def kernel(x, tok_table, pos_table):
    x2d = x.reshape(-1, _CHUNK).astype(jnp.int32)
    out = _emb(x2d, tok_table, pos_table)
    return out.reshape(_B, _T, _C)


# speedup vs baseline: 4.6146x; 4.6146x over previous
import jax
import jax.numpy as jnp
from jax import lax
from jax.experimental import pallas as pl
from jax.experimental.pallas import tpu as pltpu
from jax.experimental.pallas import tpu_sc as plsc

_B = 4
_T = 2048
_C = 64
_NC = 2
_NS = 16
_NW = _NC * _NS
_TOKS = (_B * _T) // _NW
_WPB = _NW // _B
_R = 4


def _emb_body(x_hbm, tok_hbm, pos_hbm, out_hbm,
              idx_v, cols_v, pos_v, slab0, slab1, slab2, slab3,
              sem0, sem1, sem2, sem3):
    slabs = (slab0, slab1, slab2, slab3)
    sems = (sem0, sem1, sem2, sem3)

    wid = lax.axis_index("s") * _NC + lax.axis_index("c")
    b = lax.div(wid, _WPB)
    t0 = pl.multiple_of(lax.rem(wid, _WPB) * _TOKS, 128)

    pltpu.sync_copy(x_hbm.at[pl.ds(wid * _TOKS, _TOKS)], idx_v)
    pltpu.sync_copy(pos_hbm.at[:, pl.ds(t0, _TOKS)], pos_v)

    def fire(j, idx, slot):
        start = pl.multiple_of((idx >> 7) * 128, 128)
        pltpu.async_copy(
            tok_hbm.at[:, pl.ds(start, 128)], slabs[slot], sems[slot]
        )

    def extract(j, idx, slot):
        pltpu.make_async_copy(
            tok_hbm.at[:, pl.ds(0, 128)], slabs[slot], sems[slot]
        ).wait()
        cl = jnp.full((16,), idx & 127, dtype=jnp.int32)
        cj = jnp.full((16,), j, dtype=jnp.int32)
        for jj in range(_C // 16):
            rows = lax.iota(jnp.int32, 16) + (16 * jj)
            vals = plsc.load_gather(slabs[slot], [rows, cl])
            plsc.store_scatter(cols_v, [rows, cj], vals)

    n_groups = _TOKS // 16

    def group(g, prev_chunk):
        chunk = idx_v[pl.ds(g * 16, 16)]
        for l in range(16):
            if l < _R:
                @pl.when(g >= 1)
                def _(l=l):
                    extract(g * 16 + l - _R, prev_chunk[12 + l], l % _R)
            else:
                extract(g * 16 + l - _R, chunk[l - _R], l % _R)
            fire(g * 16 + l, chunk[l], l % _R)
        return chunk

    last_chunk = lax.fori_loop(0, n_groups, group, jnp.zeros(16, jnp.int32))

    for l in range(16 - _R, 16):
        extract((n_groups - 1) * 16 + l, last_chunk[l], l % _R)

    def add_row(r, carry):
        for j in range(_TOKS // 16):
            sl = pl.ds(j * 16, 16)
            cols_v[r, sl] = cols_v[r, sl] + pos_v[r, sl]
        return carry

    lax.fori_loop(0, _C, add_row, 0)

    pltpu.sync_copy(cols_v, out_hbm.at[b, :, pl.ds(t0, _TOKS)])


@jax.jit
def _emb(x1d, tok_t, pos_t):
    mesh = plsc.VectorSubcoreMesh(core_axis_name="c", subcore_axis_name="s")
    f = pl.kernel(
        _emb_body,
        out_type=jax.ShapeDtypeStruct((_B, _C, _T), jnp.float32),
        mesh=mesh,
        scratch_types=[
            pltpu.VMEM((_TOKS,), jnp.int32),
            pltpu.VMEM((_C, _TOKS), jnp.float32),
            pltpu.VMEM((_C, _TOKS), jnp.float32),
        ] + [pltpu.VMEM((_C, 128), jnp.float32)] * _R
          + [pltpu.SemaphoreType.DMA] * _R,
        compiler_params=pltpu.CompilerParams(
            use_tc_tiling_on_sc=True, needs_layout_passes=False
        ),
    )
    return f(x1d, tok_t, pos_t)


def kernel(x, tok_table, pos_table):
    x1d = x.reshape(-1).astype(jnp.int32)
    out_t = _emb(x1d, tok_table.T, pos_table.T)
    return out_t.transpose(0, 2, 1)


# speedup vs baseline: 5.3013x; 1.1488x over previous
import jax
import jax.numpy as jnp
from jax import lax
from jax.experimental import pallas as pl
from jax.experimental.pallas import tpu as pltpu
from jax.experimental.pallas import tpu_sc as plsc

_B = 4
_T = 2048
_C = 64
_NC = 2
_NS = 16
_NW = _NC * _NS
_TOKS = (_B * _T) // _NW
_WPB = _NW // _B
_R = 8


def _emb_body(x_hbm, tok_hbm, pos_hbm, out_hbm,
              idx_v, cols_v, pos_v, *slabs_and_sems):
    slabs = slabs_and_sems[:_R]
    sems = slabs_and_sems[_R:]

    wid = lax.axis_index("s") * _NC + lax.axis_index("c")
    b = lax.div(wid, _WPB)
    t0 = pl.multiple_of(lax.rem(wid, _WPB) * _TOKS, 128)

    pltpu.sync_copy(x_hbm.at[pl.ds(wid * _TOKS, _TOKS)], idx_v)
    pltpu.sync_copy(pos_hbm.at[:, pl.ds(t0, _TOKS)], pos_v)

    def fire(j, idx, slot):
        start = pl.multiple_of((idx >> 7) * 128, 128)
        pltpu.async_copy(
            tok_hbm.at[:, pl.ds(start, 128)], slabs[slot], sems[slot]
        )

    def extract(j, idx, slot):
        pltpu.make_async_copy(
            tok_hbm.at[:, pl.ds(0, 128)], slabs[slot], sems[slot]
        ).wait()
        cl = jnp.full((16,), idx & 127, dtype=jnp.int32)
        cj = jnp.full((16,), j, dtype=jnp.int32)
        for jj in range(_C // 16):
            rows = lax.iota(jnp.int32, 16) + (16 * jj)
            vals = plsc.load_gather(slabs[slot], [rows, cl])
            plsc.store_scatter(cols_v, [rows, cj], vals)

    n_groups = _TOKS // 16

    def group(g, prev_chunk):
        chunk = idx_v[pl.ds(g * 16, 16)]
        for l in range(16):
            if l < _R:
                @pl.when(g >= 1)
                def _(l=l):
                    extract(g * 16 + l - _R, prev_chunk[16 - _R + l], l % _R)
            else:
                extract(g * 16 + l - _R, chunk[l - _R], l % _R)
            fire(g * 16 + l, chunk[l], l % _R)
        return chunk

    last_chunk = lax.fori_loop(0, n_groups, group, jnp.zeros(16, jnp.int32))

    for l in range(16 - _R, 16):
        extract((n_groups - 1) * 16 + l, last_chunk[l], l % _R)

    def add_row(r, carry):
        for j in range(_TOKS // 16):
            sl = pl.ds(j * 16, 16)
            cols_v[r, sl] = cols_v[r, sl] + pos_v[r, sl]
        return carry

    lax.fori_loop(0, _C, add_row, 0)

    pltpu.sync_copy(cols_v, out_hbm.at[b, :, pl.ds(t0, _TOKS)])


@jax.jit
def _emb(x1d, tok_t, pos_t):
    mesh = plsc.VectorSubcoreMesh(core_axis_name="c", subcore_axis_name="s")
    f = pl.kernel(
        _emb_body,
        out_type=jax.ShapeDtypeStruct((_B, _C, _T), jnp.float32),
        mesh=mesh,
        scratch_types=[
            pltpu.VMEM((_TOKS,), jnp.int32),
            pltpu.VMEM((_C, _TOKS), jnp.float32),
            pltpu.VMEM((_C, _TOKS), jnp.float32),
        ] + [pltpu.VMEM((_C, 128), jnp.float32)] * _R
          + [pltpu.SemaphoreType.DMA] * _R,
        compiler_params=pltpu.CompilerParams(
            use_tc_tiling_on_sc=True, needs_layout_passes=False
        ),
    )
    return f(x1d, tok_t, pos_t)


def kernel(x, tok_table, pos_table):
    x1d = x.reshape(-1).astype(jnp.int32)
    out_t = _emb(x1d, tok_table.T, pos_table.T)
    return out_t.transpose(0, 2, 1)


# speedup vs baseline: 5.3288x; 1.0052x over previous
import jax
import jax.numpy as jnp
from jax import lax
from jax.experimental import pallas as pl
from jax.experimental.pallas import tpu as pltpu
from jax.experimental.pallas import tpu_sc as plsc

_B = 4
_T = 2048
_C = 64
_NC = 2
_NS = 16
_NW = _NC * _NS
_TOKS = (_B * _T) // _NW
_WPB = _NW // _B
_R = 8


def _emb_body(x_hbm, tok_hbm, pos_hbm, out_hbm,
              idx_v, cols_v, pos_v, *slabs_and_sems):
    slabs = slabs_and_sems[:_R]
    sems = slabs_and_sems[_R:_R + _R]
    pos_sem = slabs_and_sems[2 * _R]

    wid = lax.axis_index("s") * _NC + lax.axis_index("c")
    b = lax.div(wid, _WPB)
    t0 = pl.multiple_of(lax.rem(wid, _WPB) * _TOKS, 128)

    pltpu.sync_copy(x_hbm.at[b, pl.ds(t0, _TOKS)], idx_v)
    pos_cp = pltpu.async_copy(pos_hbm.at[:, pl.ds(t0, _TOKS)], pos_v, pos_sem)

    def fire(j, idx, slot):
        start = pl.multiple_of((idx >> 7) * 128, 128)
        pltpu.async_copy(
            tok_hbm.at[:, pl.ds(start, 128)], slabs[slot], sems[slot]
        )

    def extract(j, idx, slot):
        pltpu.make_async_copy(
            tok_hbm.at[:, pl.ds(0, 128)], slabs[slot], sems[slot]
        ).wait()
        cl = jnp.full((16,), idx & 127, dtype=jnp.int32)
        cj = jnp.full((16,), j, dtype=jnp.int32)
        for jj in range(_C // 16):
            rows = lax.iota(jnp.int32, 16) + (16 * jj)
            vals = plsc.load_gather(slabs[slot], [rows, cl])
            plsc.store_scatter(cols_v, [rows, cj], vals)

    n_groups = _TOKS // 16

    def group(g, prev_chunk):
        chunk = idx_v[pl.ds(g * 16, 16)]
        for l in range(16):
            if l < _R:
                @pl.when(g >= 1)
                def _(l=l):
                    extract(g * 16 + l - _R, prev_chunk[16 - _R + l], l % _R)
            else:
                extract(g * 16 + l - _R, chunk[l - _R], l % _R)
            fire(g * 16 + l, chunk[l], l % _R)
        return chunk

    last_chunk = lax.fori_loop(0, n_groups, group, jnp.zeros(16, jnp.int32))

    for l in range(16 - _R, 16):
        extract((n_groups - 1) * 16 + l, last_chunk[l], l % _R)

    pos_cp.wait()

    def add_row(r, carry):
        for j in range(_TOKS // 16):
            sl = pl.ds(j * 16, 16)
            cols_v[r, sl] = cols_v[r, sl] + pos_v[r, sl]
        return carry

    lax.fori_loop(0, _C, add_row, 0)

    pltpu.sync_copy(cols_v, out_hbm.at[b, :, pl.ds(t0, _TOKS)])


@jax.jit
def _emb(x2d, tok_t, pos_t):
    mesh = plsc.VectorSubcoreMesh(core_axis_name="c", subcore_axis_name="s")
    f = pl.kernel(
        _emb_body,
        out_type=jax.ShapeDtypeStruct((_B, _C, _T), jnp.float32),
        mesh=mesh,
        scratch_types=[
            pltpu.VMEM((_TOKS,), jnp.int32),
            pltpu.VMEM((_C, _TOKS), jnp.float32),
            pltpu.VMEM((_C, _TOKS), jnp.float32),
        ] + [pltpu.VMEM((_C, 128), jnp.float32)] * _R
          + [pltpu.SemaphoreType.DMA] * (_R + 1),
        compiler_params=pltpu.CompilerParams(
            use_tc_tiling_on_sc=True, needs_layout_passes=False
        ),
    )
    return f(x2d, tok_t, pos_t)


def kernel(x, tok_table, pos_table):
    out_t = _emb(x.astype(jnp.int32), tok_table.T, pos_table.T)
    return out_t.transpose(0, 2, 1)
